# Initial kernel scaffold; baseline (speedup 1.0000x reference)
import jax
import jax.numpy as jnp
from jax.experimental import pallas as pl


def kernel(w1, b1, w2, b2, w1p, bf1, w2p, bf2, x):
    raise NotImplementedError("write your pallas kernel here")


# speedup vs baseline: 12.1713x; 12.1713x over previous
import jax
import jax.numpy as jnp
from jax.experimental import pallas as pl
from jax.experimental.pallas import tpu as pltpu

PITCH1 = 32
NROW1 = 30
ROWS1 = NROW1 * PITCH1
PITCH2 = 16
ROWS2 = 15 * PITCH2
PK = 32
GRP = 4
C1, C2 = 32, 64
FC1, NCLS = 128, 10


def _shift(a, delta):
    n = a.shape[0]
    d = (-delta) % n
    return pltpu.roll(a, d, axis=0) if d else a


def _conv_kernel(x_ref, w1_ref, b1_ref, w2_ref, b2_ref, out_ref, *scr):
    m1b = scr[:8]
    p2_s = scr[8]
    m2b = scr[9:]
    x = x_ref[...]

    cols = (_shift(x, -1), x, _shift(x, 1))
    pieces = []
    for kh in range(3):
        for kw in range(3):
            pieces.append(_shift(cols[kw], PITCH1 * (kh - 1)))
    x9 = jnp.concatenate(pieces, axis=1)
    r1 = jnp.dot(x9, w1_ref[...], preferred_element_type=jnp.float32)
    r1 = jnp.maximum(r1 + b1_ref[...], 0.0)

    a = jnp.maximum(r1, _shift(r1, 1))
    m1 = jnp.maximum(a, _shift(a, PITCH1))
    for c in range(8):
        m1b[c][...] = m1[:, 128 * c:128 * (c + 1)]

    for i in range(14):
        for c in range(8):
            p2_s[pl.ds(PITCH2 * i, PITCH2), 128 * c:128 * (c + 1)] = \
                m1b[c][pl.ds(2 * PITCH1 * i + PITCH1 - 1, PITCH2, 2), :]
    p2_s[pl.ds(14 * PITCH2, PITCH2), :] = jnp.zeros(
        (PITCH2, PK * C1), jnp.float32)
    jj = jax.lax.broadcasted_iota(jnp.int32, (ROWS2, PK * C1), 0) % PITCH2
    p2 = jnp.where((jj >= 1) & (jj <= 14), p2_s[...], 0.0)

    dcols = (_shift(p2, -1), p2, _shift(p2, 1))
    w2 = w2_ref[...]
    b2 = b2_ref[...]
    for u in range(PK // GRP):
        lo = GRP * C1 * u
        pieces = []
        for kh in range(3):
            for kw in range(3):
                blk = dcols[kw][:, lo:lo + GRP * C1]
                pieces.append(_shift(blk, PITCH2 * (kh - 1)))
        x9u = jnp.concatenate(pieces, axis=1)
        r2 = jnp.dot(x9u, w2, preferred_element_type=jnp.float32)
        r2 = jnp.maximum(r2 + b2, 0.0)
        a2 = jnp.maximum(r2, _shift(r2, 1))
        m2u = jnp.maximum(a2, _shift(a2, PITCH2))
        m2b[2 * u][...] = m2u[:, :128]
        m2b[2 * u + 1][...] = m2u[:, 128:]

    for t in range(16):
        for oh in range(7):
            v = m2b[t][pl.ds(2 * PITCH2 * oh + 1, 7, 2), :]
            out_ref[pl.ds(7 * oh, 7), 2 * t, :] = v[:, :C2]
            out_ref[pl.ds(7 * oh, 7), 2 * t + 1, :] = v[:, C2:]


def _fc_kernel(f_ref, w1_ref, b1_ref, w2_ref, b2_ref, out_ref):
    tb = f_ref.shape[1]
    acc = jnp.zeros((tb, FC1), jnp.float32)
    for p in range(49):
        acc = acc + jnp.dot(f_ref[p], w1_ref[p],
                            preferred_element_type=jnp.float32)
    h = jnp.maximum(acc + b1_ref[...], 0.0)
    out_ref[...] = (jnp.dot(h, w2_ref[...],
                            preferred_element_type=jnp.float32)
                    + b2_ref[...])


def kernel(w1, b1, w2, b2, w1p, bf1, w2p, bf2, x):
    B = x.shape[0]
    Bp = ((B + PK - 1) // PK) * PK
    G = Bp // PK

    xi = x[:, 0].astype(jnp.float32)
    if Bp != B:
        xi = jnp.pad(xi, ((0, Bp - B), (0, 0), (0, 0)))
    xp = jnp.pad(xi, ((0, 0), (1, 1), (1, 3)))
    xpk = (xp.reshape(G, PK, ROWS1).transpose(0, 2, 1)
           .reshape(G * ROWS1, PK))

    eye_pk = jnp.eye(PK, dtype=jnp.float32)
    w1m = (eye_pk[None, :, :, None] * w1.reshape(9, 1, 1, C1)
           ).reshape(9 * PK, PK * C1)
    b1m = jnp.tile(b1, (1, PK))
    eye_g = jnp.eye(GRP, dtype=jnp.float32)
    w2m = (eye_g[None, :, None, :, None] * w2.reshape(9, 1, C1, 1, C2)
           ).reshape(9 * GRP * C1, GRP * C2)
    b2m = jnp.tile(b2, (1, GRP))

    feats = pl.pallas_call(
        _conv_kernel,
        out_shape=jax.ShapeDtypeStruct((49, Bp, C2), jnp.float32),
        grid=(G,),
        in_specs=[
            pl.BlockSpec((ROWS1, PK), lambda i: (i, 0)),
            pl.BlockSpec((9 * PK, PK * C1), lambda i: (0, 0)),
            pl.BlockSpec((1, PK * C1), lambda i: (0, 0)),
            pl.BlockSpec((9 * GRP * C1, GRP * C2), lambda i: (0, 0)),
            pl.BlockSpec((1, GRP * C2), lambda i: (0, 0)),
        ],
        out_specs=pl.BlockSpec((49, PK, C2), lambda i: (0, i, 0)),
        scratch_shapes=([pltpu.VMEM((ROWS1, 128), jnp.float32)] * 8
                        + [pltpu.VMEM((ROWS2, PK * C1), jnp.float32)]
                        + [pltpu.VMEM((ROWS2, 128), jnp.float32)] * 16),
        compiler_params=pltpu.CompilerParams(
            dimension_semantics=("parallel",),
            vmem_limit_bytes=64 * 1024 * 1024),
    )(xpk, w1m, b1m, w2m, b2m)

    TB = 256
    Bfc = ((Bp + TB - 1) // TB) * TB
    if Bfc != Bp:
        feats = jnp.pad(feats, ((0, 0), (0, Bfc - Bp), (0, 0)))

    logits = pl.pallas_call(
        _fc_kernel,
        out_shape=jax.ShapeDtypeStruct((Bfc, FC1), jnp.float32),
        grid=(Bfc // TB,),
        in_specs=[
            pl.BlockSpec((49, TB, C2), lambda i: (0, i, 0)),
            pl.BlockSpec((49, C2, FC1), lambda i: (0, 0, 0)),
            pl.BlockSpec((1, FC1), lambda i: (0, 0)),
            pl.BlockSpec((FC1, FC1), lambda i: (0, 0)),
            pl.BlockSpec((1, FC1), lambda i: (0, 0)),
        ],
        out_specs=pl.BlockSpec((TB, FC1), lambda i: (i, 0)),
        compiler_params=pltpu.CompilerParams(
            dimension_semantics=("parallel",),
            vmem_limit_bytes=64 * 1024 * 1024),
    )(feats, w1p.reshape(49, C2, FC1), bf1, w2p, bf2)

    return logits[:B, :NCLS]
